# baseline (speedup 1.0000x reference)
import functools

import jax
import jax.numpy as jnp
from jax import lax
from jax.experimental import pallas as pl
from jax.experimental.pallas import tpu as pltpu
from jax.experimental.pallas import tpu_sc as plsc

_N = 10000
_D = 128
_E = 320000
_P = 16384

_NC = 2
_NS = 16
_NW = _NC * _NS
_L = 128
_HD = _D // 2
_KJ = 159
_NB = 3
_EPAD = _NS * _KJ * _L
_NP = 10240
_RPT = _NP // _NS
_PJ = (2 * _P) // (_NW * _L)


def _sc_agg_body(h_hbm, sidx_hbm, didx_hbm, zo_hbm, agg_out, cnt_out,
                 sidx_v, didx_v, r0, r1, r2, zo_v, acc_sh, cnt_sh,
                 g0, g1, g2):
    cid = lax.axis_index("c")
    sid = lax.axis_index("s")
    bufs = (r0, r1, r2)
    gsems = (g0, g1, g2)
    rows_v = r0

    pltpu.sync_copy(zo_hbm, zo_v)
    pltpu.sync_copy(zo_v.at[pl.ds(0, _RPT)],
                    cnt_sh.at[pl.ds(sid * _RPT, _RPT)])

    def _zrow(r, c):
        for k in range(_HD // 16):
            rows_v[r, pl.ds(k * 16, 16)] = jnp.zeros((16,), jnp.float32)
        return c
    lax.fori_loop(0, _L, _zrow, 0)
    for b in range(_RPT // _L):
        pltpu.sync_copy(rows_v, acc_sh.at[pl.ds(sid * _RPT + b * _L, _L)])

    pltpu.sync_copy(sidx_hbm.at[cid, sid], sidx_v)
    pltpu.sync_copy(didx_hbm.at[sid], didx_v)
    plsc.subcore_barrier()

    def _gather(j, b):
        return pltpu.make_async_copy(h_hbm.at[sidx_v.at[j]], bufs[b],
                                     gsems[b])

    def _consume(j, b):
        _gather(j, b).wait()
        pltpu.sync_copy(bufs[b], acc_sh.at[didx_v.at[j]], add=True)
        pltpu.sync_copy(zo_v.at[pl.ds(_RPT, _L)], cnt_sh.at[didx_v.at[j]],
                        add=True)

    for b in range(_NB):
        _gather(b, b).start()

    def _outer(g, c):
        for b in range(_NB):
            j = g * _NB + b
            _consume(j, b)
            _gather(j + _NB, b).start()
        return c
    lax.fori_loop(0, _KJ // _NB - 1, _outer, 0)
    for b in range(_NB):
        _consume(_KJ - _NB + b, b)
    plsc.subcore_barrier()

    pltpu.sync_copy(acc_sh.at[pl.ds(sid * _RPT, _RPT)],
                    agg_out.at[cid, pl.ds(sid * _RPT, _RPT)])

    @pl.when(cid == 0)
    def _():
        pltpu.sync_copy(cnt_sh.at[pl.ds(sid * _RPT, _RPT)],
                        cnt_out.at[pl.ds(sid * _RPT, _RPT)])


_agg_call = pl.kernel(
    _sc_agg_body,
    out_type=[
        jax.ShapeDtypeStruct((_NC, _NP, _HD), jnp.float32),
        jax.ShapeDtypeStruct((_NP, 8), jnp.float32),
    ],
    mesh=plsc.VectorSubcoreMesh(core_axis_name="c", subcore_axis_name="s"),
    scratch_types=[
        pltpu.VMEM((_KJ, _L), jnp.int32),
        pltpu.VMEM((_KJ, _L), jnp.int32),
        pltpu.VMEM((_L, _HD), jnp.float32),
        pltpu.VMEM((_L, _HD), jnp.float32),
        pltpu.VMEM((_L, _HD), jnp.float32),
        pltpu.VMEM((_RPT + _L, 8), jnp.float32),
        pltpu.VMEM_SHARED((_NP, _HD), jnp.float32),
        pltpu.VMEM_SHARED((_NP, 8), jnp.float32),
        pltpu.SemaphoreType.DMA,
        pltpu.SemaphoreType.DMA,
        pltpu.SemaphoreType.DMA,
    ],
    compiler_params=pltpu.CompilerParams(use_tc_tiling_on_sc=False),
)


def _sc_prod_body(h_hbm, sidx_hbm, didx_hbm, z_out, sidx_v, didx_v,
                  a_v, b_v, sem):
    cid = lax.axis_index("c")
    sid = lax.axis_index("s")
    wid = cid * _NS + sid
    pltpu.sync_copy(sidx_hbm.at[wid], sidx_v)
    pltpu.sync_copy(didx_hbm.at[wid], didx_v)

    def _grp(j, c):
        ca = pltpu.make_async_copy(h_hbm.at[sidx_v.at[j]], a_v, sem)
        cb = pltpu.make_async_copy(h_hbm.at[didx_v.at[j]], b_v, sem)
        ca.start()
        cb.start()
        ca.wait()
        cb.wait()

        def _mul(r, cc):
            for k in range(8):
                s = pl.ds(k * 16, 16)
                a_v[r, s] = a_v[r, s] * b_v[r, s]
            return cc
        lax.fori_loop(0, _L, _mul, 0)
        pltpu.sync_copy(a_v, z_out.at[pl.ds((wid * _PJ + j) * _L, _L)])
        return c
    lax.fori_loop(0, _PJ, _grp, 0)


_prod_call = pl.kernel(
    _sc_prod_body,
    out_type=jax.ShapeDtypeStruct((2 * _P, _D), jnp.float32),
    mesh=plsc.VectorSubcoreMesh(core_axis_name="c", subcore_axis_name="s"),
    scratch_types=[
        pltpu.VMEM((_PJ, _L), jnp.int32),
        pltpu.VMEM((_PJ, _L), jnp.int32),
        pltpu.VMEM((_L, _D), jnp.float32),
        pltpu.VMEM((_L, _D), jnp.float32),
        pltpu.SemaphoreType.DMA,
    ],
)


def _tc_sage_body(h_ref, agg_ref, cnt_ref, ws_ref, wn_ref, b_ref, fl_ref,
                  out_ref):
    agg = jnp.concatenate([agg_ref[0], agg_ref[1]], axis=1)
    cnt = cnt_ref[:, 0:1]
    mean = agg / jnp.maximum(cnt, 1.0)
    y = (jnp.dot(h_ref[...], ws_ref[...], preferred_element_type=jnp.float32)
         + jnp.dot(mean, wn_ref[...], preferred_element_type=jnp.float32)
         + b_ref[...])
    out_ref[...] = jnp.where(fl_ref[0, 0] > 0.0, jnp.maximum(y, 0.0), y)


def _tc_sage(h, agg, cnt, ws, wn, b, fl):
    r = 2000
    return pl.pallas_call(
        _tc_sage_body,
        grid=(_N // r,),
        in_specs=[
            pl.BlockSpec((r, _D), lambda i: (i, 0)),
            pl.BlockSpec((_NC, r, _HD), lambda i: (0, i, 0)),
            pl.BlockSpec((r, 8), lambda i: (i, 0)),
            pl.BlockSpec((_D, _D), lambda i: (0, 0)),
            pl.BlockSpec((_D, _D), lambda i: (0, 0)),
            pl.BlockSpec((1, _D), lambda i: (0, 0)),
            pl.BlockSpec((1, 1), lambda i: (0, 0)),
        ],
        out_specs=pl.BlockSpec((r, _D), lambda i: (i, 0)),
        out_shape=jax.ShapeDtypeStruct((_N, _D), jnp.float32),
    )(h, agg, cnt, ws, wn, b.reshape(1, _D), fl.reshape(1, 1))


def _tc_pred_body(z_ref, w1_ref, b1_ref, w2_ref, b2_ref, o_ref):
    t = jnp.dot(z_ref[...], w1_ref[...], preferred_element_type=jnp.float32)
    t = jnp.maximum(t + b1_ref[...], 0.0)
    o_ref[...] = (jnp.dot(t, w2_ref[...], preferred_element_type=jnp.float32)
                  + b2_ref[...])


def _tc_pred(z, w1, b1, w2, b2):
    r = 4096
    return pl.pallas_call(
        _tc_pred_body,
        grid=((2 * _P) // r,),
        in_specs=[
            pl.BlockSpec((r, _D), lambda i: (i, 0)),
            pl.BlockSpec((_D, _D), lambda i: (0, 0)),
            pl.BlockSpec((1, _D), lambda i: (0, 0)),
            pl.BlockSpec((_D, 1), lambda i: (0, 0)),
            pl.BlockSpec((1, 1), lambda i: (0, 0)),
        ],
        out_specs=pl.BlockSpec((r, 1), lambda i: (i, 0)),
        out_shape=jax.ShapeDtypeStruct((2 * _P, 1), jnp.float32),
    )(z, w1, b1.reshape(1, _D), w2, b2.reshape(1, 1))


def kernel(x, edge_index, pos_edges, neg_edges,
           W_self1, W_neigh1, b1, W_self2, W_neigh2, b2,
           Wp1, bp1, Wp2, bp2):
    pad = _EPAD - _E
    src_p = jnp.concatenate(
        [edge_index[0], jnp.zeros((pad,), jnp.int32)]).reshape(_NS, _KJ, _L)
    sidx = jnp.stack([2 * src_p, 2 * src_p + 1])
    pad_dst = _N + jnp.arange(pad, dtype=jnp.int32) % (_NP - _N)
    zo = jnp.concatenate([jnp.zeros((_RPT, 8), jnp.float32),
                          jnp.ones((_L, 8), jnp.float32)])
    didx = jnp.concatenate(
        [edge_index[1], pad_dst]).reshape(_NS, _KJ, _L)

    ws_s = jnp.stack([W_self1, W_self2])
    wn_s = jnp.stack([W_neigh1, W_neigh2])
    b_s = jnp.stack([b1, b2])

    n_iter = 2 + jnp.minimum(edge_index[0, 0], 0)

    def _cond(carry):
        i, _ = carry
        return i < n_iter

    def _layer(carry):
        i, h = carry
        ws = lax.dynamic_index_in_dim(ws_s, i, keepdims=False)
        wn = lax.dynamic_index_in_dim(wn_s, i, keepdims=False)
        b = lax.dynamic_index_in_dim(b_s, i, keepdims=False)
        fl = jnp.where(i == 0, 1.0, 0.0).astype(jnp.float32)
        agg, cnt = _agg_call(h.reshape(2 * _N, _HD), sidx, didx, zo)
        return i + 1, _tc_sage(h, agg, cnt, ws, wn, b, fl)

    _, h2 = lax.while_loop(_cond, _layer, (jnp.int32(0), x))

    esrc = jnp.concatenate([pos_edges[0], neg_edges[0]]).reshape(_NW, _PJ, _L)
    edst = jnp.concatenate([pos_edges[1], neg_edges[1]]).reshape(_NW, _PJ, _L)
    z = _prod_call(h2, esrc, edst)
    scores = _tc_pred(z, Wp1, bp1, Wp2, bp2)
    return (scores[:_P], scores[_P:])


# speedup vs baseline: 1.1346x; 1.1346x over previous
import functools

import jax
import jax.numpy as jnp
from jax import lax
from jax.experimental import pallas as pl
from jax.experimental.pallas import tpu as pltpu
from jax.experimental.pallas import tpu_sc as plsc

_N = 10000
_D = 128
_E = 320000
_P = 16384

_NC = 2
_NS = 16
_NW = _NC * _NS
_L = 128
_HD = _D // 2
_KJ = 158
_NB = 2
_EPAD = _NS * _KJ * _L
_NP = 10240
_RPT = _NP // _NS
_PJ = (2 * _P) // (_NW * _L)


def _sc_agg_body(h_hbm, sidx_hbm, didx_hbm, zo_hbm, agg_out, cnt_out,
                 sidx_v, didx_v, r0, r1, zo_v, acc_sh, cnt_sh,
                 g0, g1):
    cid = lax.axis_index("c")
    sid = lax.axis_index("s")
    bufs = (r0, r1)
    gsems = (g0, g1)
    rows_v = r0

    pltpu.sync_copy(zo_hbm, zo_v)
    pltpu.sync_copy(zo_v.at[pl.ds(0, _RPT)],
                    cnt_sh.at[pl.ds(sid * _RPT, _RPT)])

    def _zrow(r, c):
        for k in range(_HD // 16):
            rows_v[r, pl.ds(k * 16, 16)] = jnp.zeros((16,), jnp.float32)
        return c
    lax.fori_loop(0, _L, _zrow, 0)
    for b in range(_RPT // _L):
        pltpu.sync_copy(rows_v, acc_sh.at[pl.ds(sid * _RPT + b * _L, _L)])

    pltpu.sync_copy(sidx_hbm.at[cid, sid], sidx_v)
    pltpu.sync_copy(didx_hbm.at[sid], didx_v)
    plsc.subcore_barrier()

    def _gather(j, b):
        return pltpu.make_async_copy(h_hbm.at[sidx_v.at[j]], bufs[b],
                                     gsems[b])

    def _consume(j, b):
        _gather(j, b).wait()
        pltpu.sync_copy(bufs[b], acc_sh.at[didx_v.at[j]], add=True)
        pltpu.sync_copy(zo_v.at[pl.ds(_RPT, _L)], cnt_sh.at[didx_v.at[j]],
                        add=True)

    for b in range(_NB):
        _gather(b, b).start()

    def _outer(g, c):
        for b in range(_NB):
            j = g * _NB + b
            _consume(j, b)
            _gather(j + _NB, b).start()
        return c
    lax.fori_loop(0, _KJ // _NB - 1, _outer, 0)
    for b in range(_NB):
        _consume(_KJ - _NB + b, b)
    plsc.subcore_barrier()

    pltpu.sync_copy(acc_sh.at[pl.ds(sid * _RPT, _RPT)],
                    agg_out.at[cid, pl.ds(sid * _RPT, _RPT)])

    @pl.when(cid == 0)
    def _():
        pltpu.sync_copy(cnt_sh.at[pl.ds(sid * _RPT, _RPT)],
                        cnt_out.at[pl.ds(sid * _RPT, _RPT)])


_agg_call = pl.kernel(
    _sc_agg_body,
    out_type=[
        jax.ShapeDtypeStruct((_NC, _NP, _HD), jnp.float32),
        jax.ShapeDtypeStruct((_NP, 8), jnp.float32),
    ],
    mesh=plsc.VectorSubcoreMesh(core_axis_name="c", subcore_axis_name="s"),
    scratch_types=[
        pltpu.VMEM((_KJ, _L), jnp.int32),
        pltpu.VMEM((_KJ, _L), jnp.int32),
        pltpu.VMEM((_L, _HD), jnp.float32),
        pltpu.VMEM((_L, _HD), jnp.float32),
        pltpu.VMEM((_RPT + _L, 8), jnp.float32),
        pltpu.VMEM_SHARED((_NP, _HD), jnp.float32),
        pltpu.VMEM_SHARED((_NP, 8), jnp.float32),
        pltpu.SemaphoreType.DMA,
        pltpu.SemaphoreType.DMA,
    ],
    compiler_params=pltpu.CompilerParams(use_tc_tiling_on_sc=False),
)


def _sc_prod_body(h_hbm, sidx_hbm, didx_hbm, z_out, sidx_v, didx_v,
                  a_v, b_v, sem):
    cid = lax.axis_index("c")
    sid = lax.axis_index("s")
    wid = cid * _NS + sid
    pltpu.sync_copy(sidx_hbm.at[wid], sidx_v)
    pltpu.sync_copy(didx_hbm.at[wid], didx_v)

    def _grp(j, c):
        ca = pltpu.make_async_copy(h_hbm.at[sidx_v.at[j]], a_v, sem)
        cb = pltpu.make_async_copy(h_hbm.at[didx_v.at[j]], b_v, sem)
        ca.start()
        cb.start()
        ca.wait()
        cb.wait()

        def _mul(r, cc):
            for k in range(8):
                s = pl.ds(k * 16, 16)
                a_v[r, s] = a_v[r, s] * b_v[r, s]
            return cc
        lax.fori_loop(0, _L, _mul, 0)
        pltpu.sync_copy(a_v, z_out.at[pl.ds((wid * _PJ + j) * _L, _L)])
        return c
    lax.fori_loop(0, _PJ, _grp, 0)


_prod_call = pl.kernel(
    _sc_prod_body,
    out_type=jax.ShapeDtypeStruct((2 * _P, _D), jnp.float32),
    mesh=plsc.VectorSubcoreMesh(core_axis_name="c", subcore_axis_name="s"),
    scratch_types=[
        pltpu.VMEM((_PJ, _L), jnp.int32),
        pltpu.VMEM((_PJ, _L), jnp.int32),
        pltpu.VMEM((_L, _D), jnp.float32),
        pltpu.VMEM((_L, _D), jnp.float32),
        pltpu.SemaphoreType.DMA,
    ],
)


def _tc_sage_body(h_ref, agg_ref, cnt_ref, ws_ref, wn_ref, b_ref, fl_ref,
                  out_ref):
    agg = jnp.concatenate([agg_ref[0], agg_ref[1]], axis=1)
    cnt = cnt_ref[:, 0:1]
    mean = agg / jnp.maximum(cnt, 1.0)
    y = (jnp.dot(h_ref[...], ws_ref[...], preferred_element_type=jnp.float32)
         + jnp.dot(mean, wn_ref[...], preferred_element_type=jnp.float32)
         + b_ref[...])
    out_ref[...] = jnp.where(fl_ref[0, 0] > 0.0, jnp.maximum(y, 0.0), y)


def _tc_sage(h, agg, cnt, ws, wn, b, fl):
    r = 2000
    return pl.pallas_call(
        _tc_sage_body,
        grid=(_N // r,),
        in_specs=[
            pl.BlockSpec((r, _D), lambda i: (i, 0)),
            pl.BlockSpec((_NC, r, _HD), lambda i: (0, i, 0)),
            pl.BlockSpec((r, 8), lambda i: (i, 0)),
            pl.BlockSpec((_D, _D), lambda i: (0, 0)),
            pl.BlockSpec((_D, _D), lambda i: (0, 0)),
            pl.BlockSpec((1, _D), lambda i: (0, 0)),
            pl.BlockSpec((1, 1), lambda i: (0, 0)),
        ],
        out_specs=pl.BlockSpec((r, _D), lambda i: (i, 0)),
        out_shape=jax.ShapeDtypeStruct((_N, _D), jnp.float32),
    )(h, agg, cnt, ws, wn, b.reshape(1, _D), fl.reshape(1, 1))


def _tc_pred_body(z_ref, w1_ref, b1_ref, w2_ref, b2_ref, o_ref):
    t = jnp.dot(z_ref[...], w1_ref[...], preferred_element_type=jnp.float32)
    t = jnp.maximum(t + b1_ref[...], 0.0)
    o_ref[...] = (jnp.dot(t, w2_ref[...], preferred_element_type=jnp.float32)
                  + b2_ref[...])


def _tc_pred(z, w1, b1, w2, b2):
    r = 4096
    return pl.pallas_call(
        _tc_pred_body,
        grid=((2 * _P) // r,),
        in_specs=[
            pl.BlockSpec((r, _D), lambda i: (i, 0)),
            pl.BlockSpec((_D, _D), lambda i: (0, 0)),
            pl.BlockSpec((1, _D), lambda i: (0, 0)),
            pl.BlockSpec((_D, 1), lambda i: (0, 0)),
            pl.BlockSpec((1, 1), lambda i: (0, 0)),
        ],
        out_specs=pl.BlockSpec((r, 1), lambda i: (i, 0)),
        out_shape=jax.ShapeDtypeStruct((2 * _P, 1), jnp.float32),
    )(z, w1, b1.reshape(1, _D), w2, b2.reshape(1, 1))


def kernel(x, edge_index, pos_edges, neg_edges,
           W_self1, W_neigh1, b1, W_self2, W_neigh2, b2,
           Wp1, bp1, Wp2, bp2):
    pad = _EPAD - _E
    src_p = jnp.concatenate(
        [edge_index[0], jnp.zeros((pad,), jnp.int32)]).reshape(_NS, _KJ, _L)
    sidx = jnp.stack([2 * src_p, 2 * src_p + 1])
    pad_dst = _N + jnp.arange(pad, dtype=jnp.int32) % (_NP - _N)
    zo = jnp.concatenate([jnp.zeros((_RPT, 8), jnp.float32),
                          jnp.ones((_L, 8), jnp.float32)])
    didx = jnp.concatenate(
        [edge_index[1], pad_dst]).reshape(_NS, _KJ, _L)

    ws_s = jnp.stack([W_self1, W_self2])
    wn_s = jnp.stack([W_neigh1, W_neigh2])
    b_s = jnp.stack([b1, b2])

    n_iter = 2 + jnp.minimum(edge_index[0, 0], 0)

    def _cond(carry):
        i, _ = carry
        return i < n_iter

    def _layer(carry):
        i, h = carry
        ws = lax.dynamic_index_in_dim(ws_s, i, keepdims=False)
        wn = lax.dynamic_index_in_dim(wn_s, i, keepdims=False)
        b = lax.dynamic_index_in_dim(b_s, i, keepdims=False)
        fl = jnp.where(i == 0, 1.0, 0.0).astype(jnp.float32)
        agg, cnt = _agg_call(h.reshape(2 * _N, _HD), sidx, didx, zo)
        return i + 1, _tc_sage(h, agg, cnt, ws, wn, b, fl)

    _, h2 = lax.while_loop(_cond, _layer, (jnp.int32(0), x))

    esrc = jnp.concatenate([pos_edges[0], neg_edges[0]]).reshape(_NW, _PJ, _L)
    edst = jnp.concatenate([pos_edges[1], neg_edges[1]]).reshape(_NW, _PJ, _L)
    z = _prod_call(h2, esrc, edst)
    scores = _tc_pred(z, Wp1, bp1, Wp2, bp2)
    return (scores[:_P], scores[_P:])


# speedup vs baseline: 1.1437x; 1.0080x over previous
import functools

import jax
import jax.numpy as jnp
from jax import lax
from jax.experimental import pallas as pl
from jax.experimental.pallas import tpu as pltpu
from jax.experimental.pallas import tpu_sc as plsc

_N = 10000
_D = 128
_E = 320000
_P = 16384

_NC = 2
_NS = 16
_NW = _NC * _NS
_L = 128
_HD = _D // 2
_KJ = 158
_NB = 2
_EPAD = _NS * _KJ * _L
_NP = 10240
_RPT = _NP // _NS
_PJ = (2 * _P) // (_NW * _L)


def _sc_agg_body(h_hbm, sidx_hbm, didx_hbm, zo_hbm, agg_out, cnt_out,
                 sidx_v, didx_v, r0, r1, zo_v, acc_sh, cnt_sh,
                 g0, g1):
    cid = lax.axis_index("c")
    sid = lax.axis_index("s")
    bufs = (r0, r1)
    gsems = (g0, g1)
    rows_v = r0

    pltpu.sync_copy(zo_hbm, zo_v)
    pltpu.sync_copy(zo_v.at[pl.ds(0, _RPT)],
                    cnt_sh.at[pl.ds(sid * _RPT, _RPT)])

    def _zrow(r, c):
        for k in range(_HD // 16):
            rows_v[r, pl.ds(k * 16, 16)] = jnp.zeros((16,), jnp.float32)
        return c
    lax.fori_loop(0, _L, _zrow, 0)
    for b in range(_RPT // _L):
        pltpu.sync_copy(rows_v, acc_sh.at[pl.ds(sid * _RPT + b * _L, _L)])

    pltpu.sync_copy(sidx_hbm.at[cid, sid], sidx_v)
    pltpu.sync_copy(didx_hbm.at[sid], didx_v)
    plsc.subcore_barrier()

    def _gather(j, b):
        return pltpu.make_async_copy(h_hbm.at[sidx_v.at[j]], bufs[b],
                                     gsems[b])

    def _consume(j, b):
        _gather(j, b).wait()
        pltpu.sync_copy(bufs[b], acc_sh.at[didx_v.at[j]], add=True)
        pltpu.sync_copy(zo_v.at[pl.ds(_RPT, _L)], cnt_sh.at[didx_v.at[j]],
                        add=True)

    for b in range(_NB):
        _gather(b, b).start()

    def _outer(g, c):
        for b in range(_NB):
            j = g * _NB + b
            _consume(j, b)
            _gather(j + _NB, b).start()
        return c
    lax.fori_loop(0, _KJ // _NB - 1, _outer, 0)
    for b in range(_NB):
        _consume(_KJ - _NB + b, b)
    plsc.subcore_barrier()

    pltpu.sync_copy(acc_sh.at[pl.ds(sid * _RPT, _RPT)],
                    agg_out.at[cid, pl.ds(sid * _RPT, _RPT)])

    @pl.when(cid == 0)
    def _():
        pltpu.sync_copy(cnt_sh.at[pl.ds(sid * _RPT, _RPT)],
                        cnt_out.at[pl.ds(sid * _RPT, _RPT)])


_agg_call = pl.kernel(
    _sc_agg_body,
    out_type=[
        jax.ShapeDtypeStruct((_NC, _NP, _HD), jnp.float32),
        jax.ShapeDtypeStruct((_NP, 8), jnp.float32),
    ],
    mesh=plsc.VectorSubcoreMesh(core_axis_name="c", subcore_axis_name="s"),
    scratch_types=[
        pltpu.VMEM((_KJ, _L), jnp.int32),
        pltpu.VMEM((_KJ, _L), jnp.int32),
        pltpu.VMEM((_L, _HD), jnp.float32),
        pltpu.VMEM((_L, _HD), jnp.float32),
        pltpu.VMEM((_RPT + _L, 8), jnp.float32),
        pltpu.VMEM_SHARED((_NP, _HD), jnp.float32),
        pltpu.VMEM_SHARED((_NP, 8), jnp.float32),
        pltpu.SemaphoreType.DMA,
        pltpu.SemaphoreType.DMA,
    ],
    compiler_params=pltpu.CompilerParams(use_tc_tiling_on_sc=False),
)


def _sc_prod_body(h_hbm, sidx_hbm, didx_hbm, z_out, sidx_v, didx_v,
                  a_v, b_v, sem):
    cid = lax.axis_index("c")
    sid = lax.axis_index("s")
    wid = cid * _NS + sid
    pltpu.sync_copy(sidx_hbm.at[wid], sidx_v)
    pltpu.sync_copy(didx_hbm.at[wid], didx_v)

    def _grp(j, c):
        ca = pltpu.make_async_copy(h_hbm.at[sidx_v.at[j]], a_v, sem)
        cb = pltpu.make_async_copy(h_hbm.at[didx_v.at[j]], b_v, sem)
        ca.start()
        cb.start()
        ca.wait()
        cb.wait()

        def _mul(r, cc):
            for k in range(8):
                s = pl.ds(k * 16, 16)
                a_v[r, s] = a_v[r, s] * b_v[r, s]
            return cc
        lax.fori_loop(0, _L, _mul, 0)
        pltpu.sync_copy(a_v, z_out.at[pl.ds((wid * _PJ + j) * _L, _L)])
        return c
    lax.fori_loop(0, _PJ, _grp, 0)


_prod_call = pl.kernel(
    _sc_prod_body,
    out_type=jax.ShapeDtypeStruct((2 * _P, _D), jnp.float32),
    mesh=plsc.VectorSubcoreMesh(core_axis_name="c", subcore_axis_name="s"),
    scratch_types=[
        pltpu.VMEM((_PJ, _L), jnp.int32),
        pltpu.VMEM((_PJ, _L), jnp.int32),
        pltpu.VMEM((_L, _D), jnp.float32),
        pltpu.VMEM((_L, _D), jnp.float32),
        pltpu.SemaphoreType.DMA,
    ],
)


def _tc_sage_body(h_ref, agg_ref, cnt_ref, ws_ref, wn_ref, b_ref, fl_ref,
                  out_ref):
    fl = fl_ref[0, 0] > 0.0
    ws = jnp.where(fl, ws_ref[0], ws_ref[1])
    wn = jnp.where(fl, wn_ref[0], wn_ref[1])
    b = jnp.where(fl, b_ref[0:1, :], b_ref[1:2, :])
    agg = jnp.concatenate([agg_ref[0], agg_ref[1]], axis=1)
    cnt = cnt_ref[:, 0:1]
    mean = agg / jnp.maximum(cnt, 1.0)
    y = (jnp.dot(h_ref[...], ws, preferred_element_type=jnp.float32)
         + jnp.dot(mean, wn, preferred_element_type=jnp.float32)
         + b)
    out_ref[...] = jnp.where(fl, jnp.maximum(y, 0.0), y)


def _tc_sage(h, agg, cnt, ws_s, wn_s, b_s, fl):
    r = 2000
    return pl.pallas_call(
        _tc_sage_body,
        grid=(_N // r,),
        in_specs=[
            pl.BlockSpec((r, _D), lambda i: (i, 0)),
            pl.BlockSpec((_NC, r, _HD), lambda i: (0, i, 0)),
            pl.BlockSpec((r, 8), lambda i: (i, 0)),
            pl.BlockSpec((2, _D, _D), lambda i: (0, 0, 0)),
            pl.BlockSpec((2, _D, _D), lambda i: (0, 0, 0)),
            pl.BlockSpec((2, _D), lambda i: (0, 0)),
            pl.BlockSpec((1, 1), lambda i: (0, 0)),
        ],
        out_specs=pl.BlockSpec((r, _D), lambda i: (i, 0)),
        out_shape=jax.ShapeDtypeStruct((_N, _D), jnp.float32),
    )(h, agg, cnt, ws_s, wn_s, b_s, fl.reshape(1, 1))


def _tc_pred_body(z_ref, w1_ref, b1_ref, w2_ref, b2_ref, o_ref):
    t = jnp.dot(z_ref[...], w1_ref[...], preferred_element_type=jnp.float32)
    t = jnp.maximum(t + b1_ref[...], 0.0)
    o_ref[...] = (jnp.dot(t, w2_ref[...], preferred_element_type=jnp.float32)
                  + b2_ref[...])


def _tc_pred(z, w1, b1, w2, b2):
    r = 4096
    return pl.pallas_call(
        _tc_pred_body,
        grid=((2 * _P) // r,),
        in_specs=[
            pl.BlockSpec((r, _D), lambda i: (i, 0)),
            pl.BlockSpec((_D, _D), lambda i: (0, 0)),
            pl.BlockSpec((1, _D), lambda i: (0, 0)),
            pl.BlockSpec((_D, 1), lambda i: (0, 0)),
            pl.BlockSpec((1, 1), lambda i: (0, 0)),
        ],
        out_specs=pl.BlockSpec((r, 1), lambda i: (i, 0)),
        out_shape=jax.ShapeDtypeStruct((2 * _P, 1), jnp.float32),
    )(z, w1, b1.reshape(1, _D), w2, b2.reshape(1, 1))


def kernel(x, edge_index, pos_edges, neg_edges,
           W_self1, W_neigh1, b1, W_self2, W_neigh2, b2,
           Wp1, bp1, Wp2, bp2):
    pad = _EPAD - _E
    src_p = jnp.concatenate(
        [edge_index[0], jnp.zeros((pad,), jnp.int32)]).reshape(_NS, _KJ, _L)
    sidx = jnp.stack([2 * src_p, 2 * src_p + 1])
    pad_dst = _N + jnp.arange(pad, dtype=jnp.int32) % (_NP - _N)
    zo = jnp.concatenate([jnp.zeros((_RPT, 8), jnp.float32),
                          jnp.ones((_L, 8), jnp.float32)])
    didx = jnp.concatenate(
        [edge_index[1], pad_dst]).reshape(_NS, _KJ, _L)

    ws_s = jnp.stack([W_self1, W_self2])
    wn_s = jnp.stack([W_neigh1, W_neigh2])
    b_s = jnp.stack([b1, b2])

    n_iter = 2 + jnp.minimum(edge_index[0, 0], 0)

    def _cond(carry):
        i, _ = carry
        return i < n_iter

    def _layer(carry):
        i, h = carry
        fl = jnp.where(i == 0, 1.0, 0.0).astype(jnp.float32)
        agg, cnt = _agg_call(h.reshape(2 * _N, _HD), sidx, didx, zo)
        return i + 1, _tc_sage(h, agg, cnt, ws_s, wn_s, b_s, fl)

    _, h2 = lax.while_loop(_cond, _layer, (jnp.int32(0), x))

    esrc = jnp.concatenate([pos_edges[0], neg_edges[0]]).reshape(_NW, _PJ, _L)
    edst = jnp.concatenate([pos_edges[1], neg_edges[1]]).reshape(_NW, _PJ, _L)
    z = _prod_call(h2, esrc, edst)
    scores = _tc_pred(z, Wp1, bp1, Wp2, bp2)
    return (scores[:_P], scores[_P:])


# speedup vs baseline: 1.1658x; 1.0194x over previous
import functools

import jax
import jax.numpy as jnp
from jax import lax
from jax.experimental import pallas as pl
from jax.experimental.pallas import tpu as pltpu
from jax.experimental.pallas import tpu_sc as plsc

_N = 10000
_D = 128
_E = 320000
_P = 16384

_NC = 2
_NS = 16
_NW = _NC * _NS
_L = 128
_HD = _D // 2
_KJ = 158
_NB = 2
_EPAD = _NS * _KJ * _L
_NP = 10240
_RPT = _NP // _NS
_PJ = (2 * _P) // (_NW * _L)


def _sc_agg_body(h_hbm, sidx_hbm, didx_hbm, zo_hbm, agg_out, cnt_out,
                 sidx_v, didx_v, r0, r1, zo_v, acc_sh, cnt_sh,
                 g0, g1):
    cid = lax.axis_index("c")
    sid = lax.axis_index("s")
    bufs = (r0, r1)
    gsems = (g0, g1)
    rows_v = r0

    pltpu.sync_copy(zo_hbm, zo_v)
    pltpu.sync_copy(zo_v.at[pl.ds(0, _RPT)],
                    cnt_sh.at[pl.ds(sid * _RPT, _RPT)])

    def _zrow(r, c):
        for k in range(_HD // 16):
            rows_v[r, pl.ds(k * 16, 16)] = jnp.zeros((16,), jnp.float32)
        return c
    lax.fori_loop(0, _L, _zrow, 0)
    for b in range(_RPT // _L):
        pltpu.sync_copy(rows_v, acc_sh.at[pl.ds(sid * _RPT + b * _L, _L)])

    pltpu.sync_copy(sidx_hbm.at[cid, sid], sidx_v)
    pltpu.sync_copy(didx_hbm.at[sid], didx_v)
    plsc.subcore_barrier()

    def _gather(j, b):
        return pltpu.make_async_copy(h_hbm.at[sidx_v.at[j]], bufs[b],
                                     gsems[b])

    def _consume(j, b):
        _gather(j, b).wait()
        pltpu.sync_copy(bufs[b], acc_sh.at[didx_v.at[j]], add=True)
        pltpu.sync_copy(zo_v.at[pl.ds(_RPT, _L)], cnt_sh.at[didx_v.at[j]],
                        add=True)

    for b in range(_NB):
        _gather(b, b).start()

    def _outer(g, c):
        for b in range(_NB):
            j = g * _NB + b
            _consume(j, b)
            _gather(j + _NB, b).start()
        return c
    lax.fori_loop(0, _KJ // _NB - 1, _outer, 0)
    for b in range(_NB):
        _consume(_KJ - _NB + b, b)
    plsc.subcore_barrier()

    pltpu.sync_copy(acc_sh.at[pl.ds(sid * _RPT, _RPT)],
                    agg_out.at[cid, pl.ds(sid * _RPT, _RPT)])

    @pl.when(cid == 0)
    def _():
        pltpu.sync_copy(cnt_sh.at[pl.ds(sid * _RPT, _RPT)],
                        cnt_out.at[pl.ds(sid * _RPT, _RPT)])


_agg_call = pl.kernel(
    _sc_agg_body,
    out_type=[
        jax.ShapeDtypeStruct((_NC, _NP, _HD), jnp.float32),
        jax.ShapeDtypeStruct((_NP, 8), jnp.float32),
    ],
    mesh=plsc.VectorSubcoreMesh(core_axis_name="c", subcore_axis_name="s"),
    scratch_types=[
        pltpu.VMEM((_KJ, _L), jnp.int32),
        pltpu.VMEM((_KJ, _L), jnp.int32),
        pltpu.VMEM((_L, _HD), jnp.float32),
        pltpu.VMEM((_L, _HD), jnp.float32),
        pltpu.VMEM((_RPT + _L, 8), jnp.float32),
        pltpu.VMEM_SHARED((_NP, _HD), jnp.float32),
        pltpu.VMEM_SHARED((_NP, 8), jnp.float32),
        pltpu.SemaphoreType.DMA,
        pltpu.SemaphoreType.DMA,
    ],
    compiler_params=pltpu.CompilerParams(use_tc_tiling_on_sc=False),
)


def _sc_prod_body(h_hbm, sidx_hbm, didx_hbm, z_out, sidx_v, didx_v,
                  a0, b0, a1, b1, s0, s1):
    cid = lax.axis_index("c")
    sid = lax.axis_index("s")
    wid = cid * _NS + sid
    abufs = (a0, a1)
    bbufs = (b0, b1)
    sems = (s0, s1)
    pltpu.sync_copy(sidx_hbm.at[wid], sidx_v)
    pltpu.sync_copy(didx_hbm.at[wid], didx_v)

    def _gathers(j, b):
        return (pltpu.make_async_copy(h_hbm.at[sidx_v.at[j]], abufs[b],
                                      sems[b]),
                pltpu.make_async_copy(h_hbm.at[didx_v.at[j]], bbufs[b],
                                      sems[b]))

    def _consume(j, b):
        ca, cb = _gathers(j, b)
        ca.wait()
        cb.wait()
        a_v, b_v = abufs[b], bbufs[b]

        def _mul(r, cc):
            for k in range(8):
                sl = pl.ds(k * 16, 16)
                a_v[r, sl] = a_v[r, sl] * b_v[r, sl]
            return cc
        lax.fori_loop(0, _L, _mul, 0)
        pltpu.sync_copy(a_v, z_out.at[pl.ds((wid * _PJ + j) * _L, _L)])

    for b in range(2):
        ca, cb = _gathers(b, b)
        ca.start()
        cb.start()

    def _outer(g, c):
        for b in range(2):
            j = g * 2 + b
            _consume(j, b)
            ca, cb = _gathers(j + 2, b)
            ca.start()
            cb.start()
        return c
    lax.fori_loop(0, _PJ // 2 - 1, _outer, 0)
    for b in range(2):
        _consume(_PJ - 2 + b, b)


_prod_call = pl.kernel(
    _sc_prod_body,
    out_type=jax.ShapeDtypeStruct((2 * _P, _D), jnp.float32),
    mesh=plsc.VectorSubcoreMesh(core_axis_name="c", subcore_axis_name="s"),
    scratch_types=[
        pltpu.VMEM((_PJ, _L), jnp.int32),
        pltpu.VMEM((_PJ, _L), jnp.int32),
        pltpu.VMEM((_L, _D), jnp.float32),
        pltpu.VMEM((_L, _D), jnp.float32),
        pltpu.VMEM((_L, _D), jnp.float32),
        pltpu.VMEM((_L, _D), jnp.float32),
        pltpu.SemaphoreType.DMA,
        pltpu.SemaphoreType.DMA,
    ],
)


def _tc_sage_body(h_ref, agg_ref, cnt_ref, ws_ref, wn_ref, b_ref, fl_ref,
                  out_ref):
    fl = fl_ref[0, 0] > 0.0
    ws = jnp.where(fl, ws_ref[0], ws_ref[1])
    wn = jnp.where(fl, wn_ref[0], wn_ref[1])
    b = jnp.where(fl, b_ref[0:1, :], b_ref[1:2, :])
    agg = jnp.concatenate([agg_ref[0], agg_ref[1]], axis=1)
    cnt = cnt_ref[:, 0:1]
    mean = agg / jnp.maximum(cnt, 1.0)
    y = (jnp.dot(h_ref[...], ws, preferred_element_type=jnp.float32)
         + jnp.dot(mean, wn, preferred_element_type=jnp.float32)
         + b)
    out_ref[...] = jnp.where(fl, jnp.maximum(y, 0.0), y)


def _tc_sage(h, agg, cnt, ws_s, wn_s, b_s, fl):
    r = 2000
    return pl.pallas_call(
        _tc_sage_body,
        grid=(_N // r,),
        in_specs=[
            pl.BlockSpec((r, _D), lambda i: (i, 0)),
            pl.BlockSpec((_NC, r, _HD), lambda i: (0, i, 0)),
            pl.BlockSpec((r, 8), lambda i: (i, 0)),
            pl.BlockSpec((2, _D, _D), lambda i: (0, 0, 0)),
            pl.BlockSpec((2, _D, _D), lambda i: (0, 0, 0)),
            pl.BlockSpec((2, _D), lambda i: (0, 0)),
            pl.BlockSpec((1, 1), lambda i: (0, 0)),
        ],
        out_specs=pl.BlockSpec((r, _D), lambda i: (i, 0)),
        out_shape=jax.ShapeDtypeStruct((_N, _D), jnp.float32),
    )(h, agg, cnt, ws_s, wn_s, b_s, fl.reshape(1, 1))


def _tc_pred_body(z_ref, w1_ref, b1_ref, w2_ref, b2_ref, o_ref):
    t = jnp.dot(z_ref[...], w1_ref[...], preferred_element_type=jnp.float32)
    t = jnp.maximum(t + b1_ref[...], 0.0)
    o_ref[...] = (jnp.dot(t, w2_ref[...], preferred_element_type=jnp.float32)
                  + b2_ref[...])


def _tc_pred(z, w1, b1, w2, b2):
    r = 8192
    return pl.pallas_call(
        _tc_pred_body,
        grid=((2 * _P) // r,),
        in_specs=[
            pl.BlockSpec((r, _D), lambda i: (i, 0)),
            pl.BlockSpec((_D, _D), lambda i: (0, 0)),
            pl.BlockSpec((1, _D), lambda i: (0, 0)),
            pl.BlockSpec((_D, 1), lambda i: (0, 0)),
            pl.BlockSpec((1, 1), lambda i: (0, 0)),
        ],
        out_specs=pl.BlockSpec((r, 1), lambda i: (i, 0)),
        out_shape=jax.ShapeDtypeStruct((2 * _P, 1), jnp.float32),
    )(z, w1, b1.reshape(1, _D), w2, b2.reshape(1, 1))


def kernel(x, edge_index, pos_edges, neg_edges,
           W_self1, W_neigh1, b1, W_self2, W_neigh2, b2,
           Wp1, bp1, Wp2, bp2):
    pad = _EPAD - _E
    src_p = jnp.concatenate(
        [edge_index[0], jnp.zeros((pad,), jnp.int32)]).reshape(_NS, _KJ, _L)
    sidx = jnp.stack([2 * src_p, 2 * src_p + 1])
    pad_dst = _N + jnp.arange(pad, dtype=jnp.int32) % (_NP - _N)
    zo = jnp.concatenate([jnp.zeros((_RPT, 8), jnp.float32),
                          jnp.ones((_L, 8), jnp.float32)])
    didx = jnp.concatenate(
        [edge_index[1], pad_dst]).reshape(_NS, _KJ, _L)

    ws_s = jnp.stack([W_self1, W_self2])
    wn_s = jnp.stack([W_neigh1, W_neigh2])
    b_s = jnp.stack([b1, b2])

    n_iter = 2 + jnp.minimum(edge_index[0, 0], 0)

    def _cond(carry):
        i, _ = carry
        return i < n_iter

    def _layer(carry):
        i, h = carry
        fl = jnp.where(i == 0, 1.0, 0.0).astype(jnp.float32)
        agg, cnt = _agg_call(h.reshape(2 * _N, _HD), sidx, didx, zo)
        return i + 1, _tc_sage(h, agg, cnt, ws_s, wn_s, b_s, fl)

    _, h2 = lax.while_loop(_cond, _layer, (jnp.int32(0), x))

    esrc = jnp.concatenate([pos_edges[0], neg_edges[0]]).reshape(_NW, _PJ, _L)
    edst = jnp.concatenate([pos_edges[1], neg_edges[1]]).reshape(_NW, _PJ, _L)
    z = _prod_call(h2, esrc, edst)
    scores = _tc_pred(z, Wp1, bp1, Wp2, bp2)
    return (scores[:_P], scores[_P:])
